# baseline (speedup 1.0000x reference)
import jax
import jax.numpy as jnp
from jax.experimental import pallas as pl
from jax.experimental.pallas import tpu as pltpu

_B = 32
_ROWS = _B * 3
_G = 12
_BPG = _G // 3


def _body(cam_s, id_s, wcam_ref, bcam_ref, wt_any, bt_any,
          wident_any, bident_any, img_ref, out_ref,
          wvs, bvs, wscr, bscr, wiscr, biscr, sem):
    i = pl.program_id(0)
    cps = []
    for t in range(_BPG):
        k = i * _BPG + t
        c8 = pl.multiple_of((cam_s[k] // 8) * 8, 8)
        i8 = pl.multiple_of((id_s[k] // 8) * 8, 8)
        l0 = pl.multiple_of((id_s[k] // 128) * 128, 128)
        cps.append(pltpu.make_async_copy(
            wt_any.at[:, pl.ds(c8, 8), pl.ds(l0, 128)], wscr.at[t], sem))
        cps.append(pltpu.make_async_copy(
            bt_any.at[:, pl.ds(c8, 8), pl.ds(l0, 128)], bscr.at[t], sem))
        cps.append(pltpu.make_async_copy(
            wident_any.at[pl.ds(i8, 8), :], wiscr.at[t], sem))
        cps.append(pltpu.make_async_copy(
            bident_any.at[pl.ds(i8, 8), :], biscr.at[t], sem))
    for cp in cps:
        cp.start()
    for cp in cps:
        cp.wait()
    sub_i = jax.lax.broadcasted_iota(jnp.int32, (8, 128), 0)
    lane_i = jax.lax.broadcasted_iota(jnp.int32, (8, 128), 1)
    sub83 = jax.lax.broadcasted_iota(jnp.int32, (8, 3), 0)
    lane3 = jax.lax.broadcasted_iota(jnp.int32, (1, 3), 1)
    for t in range(_BPG):
        k = i * _BPG + t
        ci = cam_s[k]
        ii = id_s[k]
        msk = jnp.where((sub_i == ci % 8) & (lane_i == ii % 128), 1.0, 0.0)
        m83 = jnp.where(sub83 == ii % 8, 1.0, 0.0)
        wcrow = wcam_ref[pl.ds(ci, 1), :]
        bcrow = bcam_ref[pl.ds(ci, 1), :]
        for c in range(3):
            one = jnp.where(lane3 == c, 1.0, 0.0)
            wvs[t, c] = (jnp.sum(wcrow * one)
                         + jnp.sum(wiscr[t] * m83 * one)
                         + 10.0 * jnp.sum(wscr[t, c] * msk))
            bvs[t, c] = (jnp.sum(bcrow * one)
                         + jnp.sum(biscr[t] * m83 * one)
                         + 10.0 * jnp.sum(bscr[t, c] * msk))
    for j in range(_G):
        t = j // 3
        c = j - 3 * t
        out_ref[j] = img_ref[j] * wvs[t, c] + bvs[t, c]


def kernel(image, camindex, idindex, wcam, bcam, wident, bident, w, b):
    bsz, ch, h, ww = image.shape
    cam = camindex.astype(jnp.int32)
    idn = idindex.astype(jnp.int32)
    wt = jnp.transpose(w, (2, 0, 1))
    bt = jnp.transpose(b, (2, 0, 1))
    grid_spec = pltpu.PrefetchScalarGridSpec(
        num_scalar_prefetch=2,
        grid=(_ROWS // _G,),
        in_specs=[
            pl.BlockSpec((100, 3), lambda i, cs, ids: (0, 0)),
            pl.BlockSpec((100, 3), lambda i, cs, ids: (0, 0)),
            pl.BlockSpec(memory_space=pl.ANY),
            pl.BlockSpec(memory_space=pl.ANY),
            pl.BlockSpec(memory_space=pl.ANY),
            pl.BlockSpec(memory_space=pl.ANY),
            pl.BlockSpec((_G, h, ww), lambda i, cs, ids: (i, 0, 0)),
        ],
        out_specs=pl.BlockSpec((_G, h, ww), lambda i, cs, ids: (i, 0, 0)),
        scratch_shapes=[
            pltpu.SMEM((_BPG, 3), jnp.float32),
            pltpu.SMEM((_BPG, 3), jnp.float32),
            pltpu.VMEM((_BPG, 3, 8, 128), jnp.float32),
            pltpu.VMEM((_BPG, 3, 8, 128), jnp.float32),
            pltpu.VMEM((_BPG, 8, 3), jnp.float32),
            pltpu.VMEM((_BPG, 8, 3), jnp.float32),
            pltpu.SemaphoreType.DMA,
        ],
    )
    out = pl.pallas_call(
        _body,
        grid_spec=grid_spec,
        out_shape=jax.ShapeDtypeStruct((_ROWS, h, ww), jnp.float32),
    )(cam, idn, wcam, bcam, wt, bt, wident, bident,
      image.reshape(bsz * ch, h, ww))
    return out.reshape(bsz, ch, h, ww)


# speedup vs baseline: 1.2190x; 1.2190x over previous
import jax
import jax.numpy as jnp
from jax.experimental import pallas as pl
from jax.experimental.pallas import tpu as pltpu

_B = 32
_ROWS = _B * 3
_G = 12


def _body(cam_s, id_s, wcam_ref, bcam_ref, wident_any, bident_any,
          wt_any, bt_any, img_ref, out_ref,
          wvs, bvs, wscr, bscr, wiscr, biscr, sem):
    i = pl.program_id(0)

    @pl.when(i == 0)
    def _prologue():
        cps = []
        for k in range(_B):
            c8 = pl.multiple_of((cam_s[k] // 8) * 8, 8)
            i8 = pl.multiple_of((id_s[k] // 8) * 8, 8)
            l0 = pl.multiple_of((id_s[k] // 128) * 128, 128)
            cps.append(pltpu.make_async_copy(
                wt_any.at[:, pl.ds(c8, 8), pl.ds(l0, 128)], wscr.at[k], sem))
            cps.append(pltpu.make_async_copy(
                bt_any.at[:, pl.ds(c8, 8), pl.ds(l0, 128)], bscr.at[k], sem))
            cps.append(pltpu.make_async_copy(
                wident_any.at[pl.ds(i8, 8), :], wiscr.at[k], sem))
            cps.append(pltpu.make_async_copy(
                bident_any.at[pl.ds(i8, 8), :], biscr.at[k], sem))
        for cp in cps:
            cp.start()
        for cp in cps:
            cp.wait()
        sub_i = jax.lax.broadcasted_iota(jnp.int32, (8, 128), 0)
        lane_i = jax.lax.broadcasted_iota(jnp.int32, (8, 128), 1)
        lane3 = jax.lax.broadcasted_iota(jnp.int32, (1, 3), 1)
        sub83 = jax.lax.broadcasted_iota(jnp.int32, (8, 3), 0)
        for k in range(_B):
            ci = cam_s[k]
            ii = id_s[k]
            msk = jnp.where((sub_i == ci % 8) & (lane_i == ii % 128), 1.0, 0.0)
            m83 = jnp.where(sub83 == ii % 8, 1.0, 0.0)
            wcrow = wcam_ref[pl.ds(ci, 1), :]
            bcrow = bcam_ref[pl.ds(ci, 1), :]
            for c in range(3):
                one = jnp.where(lane3 == c, 1.0, 0.0)
                wvs[k, c] = (jnp.sum(wcrow * one)
                             + jnp.sum(wiscr[k] * m83 * one)
                             + 10.0 * jnp.sum(wscr[k, c] * msk))
                bvs[k, c] = (jnp.sum(bcrow * one)
                             + jnp.sum(biscr[k] * m83 * one)
                             + 10.0 * jnp.sum(bscr[k, c] * msk))

    for j in range(_G):
        r = i * _G + j
        b = r // 3
        c = r - 3 * b
        out_ref[j] = img_ref[j] * wvs[b, c] + bvs[b, c]


def kernel(image, camindex, idindex, wcam, bcam, wident, bident, w, b):
    bsz, ch, h, ww = image.shape
    cam = camindex.astype(jnp.int32)
    idn = idindex.astype(jnp.int32)
    wt = jnp.transpose(w, (2, 0, 1))
    bt = jnp.transpose(b, (2, 0, 1))
    grid_spec = pltpu.PrefetchScalarGridSpec(
        num_scalar_prefetch=2,
        grid=(_ROWS // _G,),
        in_specs=[
            pl.BlockSpec((100, 3), lambda i, cs, ids: (0, 0)),
            pl.BlockSpec((100, 3), lambda i, cs, ids: (0, 0)),
            pl.BlockSpec(memory_space=pl.ANY),
            pl.BlockSpec(memory_space=pl.ANY),
            pl.BlockSpec(memory_space=pl.ANY),
            pl.BlockSpec(memory_space=pl.ANY),
            pl.BlockSpec((_G, h, ww), lambda i, cs, ids: (i, 0, 0)),
        ],
        out_specs=pl.BlockSpec((_G, h, ww), lambda i, cs, ids: (i, 0, 0)),
        scratch_shapes=[
            pltpu.SMEM((_B, 3), jnp.float32),
            pltpu.SMEM((_B, 3), jnp.float32),
            pltpu.VMEM((_B, 3, 8, 128), jnp.float32),
            pltpu.VMEM((_B, 3, 8, 128), jnp.float32),
            pltpu.VMEM((_B, 8, 3), jnp.float32),
            pltpu.VMEM((_B, 8, 3), jnp.float32),
            pltpu.SemaphoreType.DMA,
        ],
    )
    out = pl.pallas_call(
        _body,
        grid_spec=grid_spec,
        out_shape=jax.ShapeDtypeStruct((_ROWS, h, ww), jnp.float32),
    )(cam, idn, wcam, bcam, wident, bident, wt, bt,
      image.reshape(bsz * ch, h, ww))
    return out.reshape(bsz, ch, h, ww)
